# Initial kernel scaffold; baseline (speedup 1.0000x reference)
import jax
import jax.numpy as jnp
from jax.experimental import pallas as pl


def kernel(x, edge_index, params):
    raise NotImplementedError("write your pallas kernel here")


# speedup vs baseline: 1.0299x; 1.0299x over previous
import functools

import jax
import jax.numpy as jnp
from jax.experimental import pallas as pl
from jax.experimental.pallas import tpu as pltpu

E_BLK = 10000


def _mlp_body(n_layers, final_relu, in_ref, *rest):
    out_ref = rest[-1]
    wrefs = rest[:-1]
    h = in_ref[...]
    for i in range(n_layers):
        W = wrefs[2 * i][...]
        b = wrefs[2 * i + 1][...]
        h = jax.lax.dot_general(h, W, (((1,), (0,)), ((), ())),
                                preferred_element_type=jnp.float32) + b
        if i < n_layers - 1 or final_relu:
            h = jnp.maximum(h, 0.0)
    out_ref[...] = h


def _edge_mlp(feat, layers, final_relu):
    E = feat.shape[0]
    n_layers = len(layers)
    f_out = layers[-1][0].shape[1]
    grid = E // E_BLK
    assert grid * E_BLK == E

    w_args = []
    w_specs = []
    for (W, b) in layers:
        w_args.append(W)
        w_specs.append(pl.BlockSpec(W.shape, lambda i: (0, 0)))
        w_args.append(b.reshape(1, -1))
        w_specs.append(pl.BlockSpec((1, b.shape[0]), lambda i: (0, 0)))

    return pl.pallas_call(
        functools.partial(_mlp_body, n_layers, final_relu),
        grid=(grid,),
        in_specs=[pl.BlockSpec((E_BLK, feat.shape[1]), lambda i: (i, 0))] + w_specs,
        out_specs=pl.BlockSpec((E_BLK, f_out), lambda i: (i, 0)),
        out_shape=jax.ShapeDtypeStruct((E, f_out), jnp.float32),
    )(feat, *w_args)


def _edge_conv(h, src, dst, inv_cnt, layers, final_relu):
    x_i = jnp.take(h, dst, axis=0)
    x_j = jnp.take(h, src, axis=0)
    feat = jnp.concatenate([x_i, x_j - x_i], axis=-1)
    m = _edge_mlp(feat, layers, final_relu)
    s = jax.ops.segment_sum(m, dst, num_segments=h.shape[0])
    return s * inv_cnt[:, None]


def kernel(x, edge_index, params):
    n = x.shape[0]
    src = edge_index[0]
    dst = edge_index[1]
    cnt = jax.ops.segment_sum(jnp.ones((dst.shape[0],), jnp.float32), dst,
                              num_segments=n)
    inv_cnt = 1.0 / jnp.maximum(cnt, 1.0)

    mean = jnp.mean(x, axis=0)
    var = jnp.var(x, axis=0)
    h = (x - mean) / jnp.sqrt(var + 1e-5) * params["bn_gamma"] + params["bn_beta"]

    h = _edge_conv(h, src, dst, inv_cnt, params["enc1"], True)
    h = _edge_conv(h, src, dst, inv_cnt, params["enc2"], True)
    h = _edge_conv(h, src, dst, inv_cnt, params["dec1"], False)
    return h


# speedup vs baseline: 4.0892x; 3.9704x over previous
import functools

import jax
import jax.numpy as jnp
from jax import lax
from jax.experimental import pallas as pl
from jax.experimental.pallas import tpu as pltpu
from jax.experimental.pallas import tpu_sc as plsc

N = 100000
E = 1600000
NC, NS, L = 2, 16, 16
NW = NC * NS

GRP = 100
K_J = 8
CHUNK = GRP * K_J

E_BLK = 10000


def _gather_body(table_hbm, idxg_hbm, out_hbm, idx_v, rows_v, sem):
    wid = lax.axis_index("s") * NC + lax.axis_index("c")
    rows_per_w = (2 * E) // NW
    nchunks = rows_per_w // CHUNK

    def chunk(ci, carry):
        row0 = pl.multiple_of(wid * rows_per_w + ci * CHUNK, 8)
        grp0 = pl.multiple_of(wid * (rows_per_w // GRP) + ci * K_J, 8)
        pltpu.sync_copy(idxg_hbm.at[pl.ds(grp0, K_J)], idx_v)
        cps = []
        for j in range(K_J):
            cps.append(pltpu.async_copy(
                table_hbm.at[idx_v.at[j]],
                rows_v.at[pl.ds(j * GRP, GRP)], sem))
        for cp in cps:
            cp.wait()
        pltpu.sync_copy(rows_v, out_hbm.at[pl.ds(row0, CHUNK)])
        return carry

    lax.fori_loop(0, nchunks, chunk, 0, unroll=False)


def _sc_gather(table, idx_grp):
    mesh = plsc.VectorSubcoreMesh(core_axis_name="c", subcore_axis_name="s")
    f = pl.kernel(
        _gather_body,
        mesh=mesh,
        compiler_params=pltpu.CompilerParams(use_tc_tiling_on_sc=False),
        out_type=jax.ShapeDtypeStruct((2 * E, 16), jnp.float32),
        scratch_types=[
            pltpu.VMEM((K_J, GRP), jnp.int32),
            pltpu.VMEM((CHUNK, 16), jnp.float32),
            pltpu.SemaphoreType.DMA,
        ],
    )
    return f(table, idx_grp)


def _zero_acc(zeros_hbm, zb_v, acc, s, width):
    pltpu.sync_copy(zeros_hbm, zb_v)
    nchunks = -(-N // CHUNK)
    for i in range(-(-nchunks // NS)):
        k = i * NS + s
        @pl.when(k * CHUNK < N)
        def _():
            r0 = pl.multiple_of(k * CHUNK, 8)
            pltpu.sync_copy(zb_v, acc.at[pl.ds(r0, CHUNK)])


def _write_out(out_hbm, acc, c, s):
    for i in range(-(-(N // CHUNK) // NS)):
        k = i * NS + s
        @pl.when(k * CHUNK < N)
        def _():
            r0 = pl.multiple_of(k * CHUNK, 8)
            pltpu.sync_copy(acc.at[pl.ds(r0, CHUNK)],
                            out_hbm.at[pl.ds(c * N + r0, CHUNK)])


def _scatter_wide_body(msgs_hbm, dstg_hbm, zeros_hbm, out_hbm,
                       idx_v, rows_v, zb_v, acc, sem):
    c = lax.axis_index("c")
    s = lax.axis_index("s")
    _zero_acc(zeros_hbm, zb_v, acc, s, 16)
    plsc.subcore_barrier()

    nchunks_per_tile = E // (CHUNK * NS)

    def chunk(i, carry):
        k = i * NS + s
        grp0 = pl.multiple_of(k * K_J, 8)
        row0 = pl.multiple_of(c * E + k * CHUNK, 8)
        pltpu.sync_copy(dstg_hbm.at[pl.ds(grp0, K_J)], idx_v)
        pltpu.sync_copy(msgs_hbm.at[pl.ds(row0, CHUNK)], rows_v)
        cps = []
        for j in range(K_J):
            cps.append(pltpu.async_copy(
                rows_v.at[pl.ds(j * GRP, GRP)],
                acc.at[idx_v.at[j]], sem, add=True))
        for cp in cps:
            cp.wait()
        return carry

    lax.fori_loop(0, nchunks_per_tile, chunk, 0, unroll=False)
    plsc.subcore_barrier()
    _write_out(out_hbm, acc, c, s)


def _sc_scatter_wide(msgs_flat, dst_grp, zeros_c):
    mesh = plsc.VectorSubcoreMesh(core_axis_name="c", subcore_axis_name="s")
    f = pl.kernel(
        _scatter_wide_body,
        mesh=mesh,
        compiler_params=pltpu.CompilerParams(use_tc_tiling_on_sc=False),
        out_type=jax.ShapeDtypeStruct((2 * N, 16), jnp.float32),
        scratch_types=[
            pltpu.VMEM((K_J, GRP), jnp.int32),
            pltpu.VMEM((CHUNK, 16), jnp.float32),
            pltpu.VMEM((CHUNK, 16), jnp.float32),
            pltpu.VMEM_SHARED((N, 16), jnp.float32),
            pltpu.SemaphoreType.DMA,
        ],
    )
    return f(msgs_flat, dst_grp, zeros_c)


def _scatter_narrow_body(F, counts_mode, msgs_hbm, dstg_hbm, zeros_hbm,
                         out_hbm, idx_v, rows_v, zb_v, acc, sem):
    c = lax.axis_index("c")
    s = lax.axis_index("s")
    _zero_acc(zeros_hbm, zb_v, acc, s, F)
    if counts_mode:
        pltpu.sync_copy(msgs_hbm, rows_v)
    plsc.subcore_barrier()

    wid = s * NC + c
    total_chunks = E // CHUNK

    def chunk(i, carry):
        k = i * NW + wid

        @pl.when(k < total_chunks)
        def _():
            grp0 = pl.multiple_of(k * K_J, 8)
            pltpu.sync_copy(dstg_hbm.at[pl.ds(grp0, K_J)], idx_v)
            if not counts_mode:
                row0 = pl.multiple_of(k * CHUNK, 8)
                pltpu.sync_copy(msgs_hbm.at[pl.ds(row0, CHUNK)], rows_v)
            cps = []
            for j in range(K_J):
                src = rows_v if counts_mode else rows_v.at[pl.ds(j * GRP, GRP)]
                cps.append(pltpu.async_copy(
                    src, acc.at[idx_v.at[j]], sem, add=True))
            for cp in cps:
                cp.wait()
        return carry

    lax.fori_loop(0, -(-total_chunks // NW), chunk, 0, unroll=False)
    plsc.subcore_barrier()
    _write_out(out_hbm, acc, c, s)


def _sc_scatter_narrow(msgs, dst_grp, zeros_c, F, counts_mode=False):
    mesh = plsc.VectorSubcoreMesh(core_axis_name="c", subcore_axis_name="s")
    f = pl.kernel(
        functools.partial(_scatter_narrow_body, F, counts_mode),
        mesh=mesh,
        compiler_params=pltpu.CompilerParams(use_tc_tiling_on_sc=False),
        out_type=jax.ShapeDtypeStruct((2 * N, F), jnp.float32),
        scratch_types=[
            pltpu.VMEM((K_J, GRP), jnp.int32),
            pltpu.VMEM((GRP, F) if counts_mode else (CHUNK, F), jnp.float32),
            pltpu.VMEM((CHUNK, F), jnp.float32),
            pltpu.VMEM_SHARED((N, F), jnp.float32),
            pltpu.SemaphoreType.DMA,
        ],
    )
    return f(msgs, dst_grp, zeros_c)


def _edge_mlp_body(first_proj, final_relu, split_out, in_ref, *rest):
    out_ref = rest[-1]
    wrefs = rest[:-1]
    g_src = in_ref[0]
    g_dst = in_ref[1]

    def mat(h, w):
        return lax.dot_general(h, w, (((1,), (0,)), ((), ())),
                               preferred_element_type=jnp.float32)

    if first_proj:
        h = mat(g_src, wrefs[0][...]) + mat(g_dst, wrefs[1][...]) + wrefs[2][...]
        i = 3
    else:
        h = g_src + g_dst + wrefs[0][...]
        i = 1
    h = jnp.maximum(h, 0.0)
    n_rest = (len(wrefs) - i) // 2
    for k in range(n_rest):
        h = mat(h, wrefs[i + 2 * k][...]) + wrefs[i + 2 * k + 1][...]
        if k < n_rest - 1 or final_relu:
            h = jnp.maximum(h, 0.0)
    if split_out:
        out_ref[0] = h[:, :16]
        out_ref[1] = h[:, 16:]
    else:
        out_ref[...] = h


def _edge_mlp(g, first_proj, final_relu, wlist, f_out, split_out):
    grid = E // E_BLK
    w_specs = [pl.BlockSpec(w.shape, lambda i: (0,) * w.ndim) for w in wlist]
    if split_out:
        out_spec = pl.BlockSpec((2, E_BLK, 16), lambda i: (0, i, 0))
        out_shape = jax.ShapeDtypeStruct((2, E, 16), jnp.float32)
    else:
        out_spec = pl.BlockSpec((E_BLK, f_out), lambda i: (i, 0))
        out_shape = jax.ShapeDtypeStruct((E, f_out), jnp.float32)
    return pl.pallas_call(
        functools.partial(_edge_mlp_body, first_proj, final_relu, split_out),
        grid=(grid,),
        in_specs=[pl.BlockSpec((2, E_BLK, 16), lambda i: (0, i, 0))] + w_specs,
        out_specs=out_spec,
        out_shape=out_shape,
    )(g, *wlist)


N_BLK = 5000


def _inv_cnt_kernel(cnt2v):
    def body(c_ref, inv_ref):
        c = c_ref[0] + c_ref[1]
        inv_ref[...] = 1.0 / jnp.maximum(c, 1.0)

    return pl.pallas_call(
        body,
        out_shape=jax.ShapeDtypeStruct((1000, 100), jnp.float32),
    )(cnt2v)


def _bn_table(x32, gamma, beta):
    import numpy as _np
    Gn = _np.zeros((32, 4), _np.float32)
    for i in range(32):
        Gn[i, i % 4] = 1.0
    G = jnp.asarray(Gn)

    def stats_body(x_ref, g_ref, gm_ref, bt_ref, sc_ref, sh_ref):
        xv = x_ref[...]
        g = g_ref[...]
        s = jnp.sum(xv, axis=0, keepdims=True)
        mean4 = lax.dot_general(s, g, (((1,), (0,)), ((), ())),
                                preferred_element_type=jnp.float32) / N
        mean32 = lax.dot_general(mean4, g, (((1,), (1,)), ((), ())),
                                 preferred_element_type=jnp.float32)
        xc = xv - mean32
        v = jnp.sum(xc * xc, axis=0, keepdims=True)
        var4 = lax.dot_general(v, g, (((1,), (0,)), ((), ())),
                               preferred_element_type=jnp.float32) / N
        rs4 = gm_ref[...] * jax.lax.rsqrt(var4 + 1e-5)
        sh4 = bt_ref[...] - mean4 * rs4
        sc_ref[...] = lax.dot_general(rs4, g, (((1,), (1,)), ((), ())),
                                      preferred_element_type=jnp.float32)
        sh_ref[...] = lax.dot_general(sh4, g, (((1,), (1,)), ((), ())),
                                      preferred_element_type=jnp.float32)

    scale32, shift32 = pl.pallas_call(
        stats_body,
        out_shape=(jax.ShapeDtypeStruct((1, 32), jnp.float32),
                   jax.ShapeDtypeStruct((1, 32), jnp.float32)),
    )(x32, G, gamma.reshape(1, 4), beta.reshape(1, 4))

    Sn = _np.zeros((4, 16), _np.float32)
    for f in range(4):
        Sn[f, f] = 1.0
    S = jnp.asarray(Sn)

    blk = N_BLK
    grid = N // blk

    def apply_body(x4_ref, sc4_ref, sh4_ref, s_ref, out_ref):
        h = x4_ref[...] * sc4_ref[...] + sh4_ref[...]
        t = lax.dot_general(h, s_ref[...], (((1,), (0,)), ((), ())),
                            preferred_element_type=jnp.float32)
        out_ref[0] = t
        out_ref[1] = t

    x4 = x32.reshape(N, 4)
    scale4 = scale32[:, :4]
    shift4 = shift32[:, :4]
    table = pl.pallas_call(
        apply_body,
        grid=(grid,),
        in_specs=[pl.BlockSpec((blk, 4), lambda i: (i, 0)),
                  pl.BlockSpec((1, 4), lambda i: (0, 0)),
                  pl.BlockSpec((1, 4), lambda i: (0, 0)),
                  pl.BlockSpec((4, 16), lambda i: (0, 0))],
        out_specs=pl.BlockSpec((2, blk, 16), lambda i: (0, i, 0)),
        out_shape=jax.ShapeDtypeStruct((2, N, 16), jnp.float32),
    )(x4, scale4, shift4, S)
    return table.reshape(2 * N, 16)


def _node_enc2_tables(s1, inv, Wb, Wd):
    blk = N_BLK
    grid = N // blk

    def body(a_ref, b_ref, inv_ref, wb_ref, wd_ref, out_ref):
        h1 = jnp.concatenate([a_ref[...], b_ref[...]], axis=1) * inv_ref[...]
        out_ref[0] = lax.dot_general(h1, wb_ref[...], (((1,), (0,)), ((), ())),
                                     preferred_element_type=jnp.float32)
        out_ref[1] = lax.dot_general(h1, wd_ref[...], (((1,), (0,)), ((), ())),
                                     preferred_element_type=jnp.float32)

    out = pl.pallas_call(
        body,
        grid=(grid,),
        in_specs=[pl.BlockSpec((blk, 16), lambda i: (i, 0)),
                  pl.BlockSpec((blk, 16), lambda i: (grid + i, 0)),
                  pl.BlockSpec((blk, 1), lambda i: (i, 0)),
                  pl.BlockSpec((32, 16), lambda i: (0, 0)),
                  pl.BlockSpec((32, 16), lambda i: (0, 0))],
        out_specs=pl.BlockSpec((2, blk, 16), lambda i: (0, i, 0)),
        out_shape=jax.ShapeDtypeStruct((2, N, 16), jnp.float32),
    )(s1, s1, inv, Wb, Wd)
    return out.reshape(2 * N, 16)


def _node_dec1_table(s2, inv):
    blk = N_BLK
    grid = N // blk
    import numpy as _np
    Sn = _np.zeros((8, 16), _np.float32)
    Sn[0, 0] = 1.0
    Sn[1, 1] = 1.0
    S2 = jnp.asarray(Sn)

    def body(a_ref, b_ref, inv_ref, s_ref, out_ref):
        h2 = (a_ref[...] + b_ref[...]) * inv_ref[...]
        t = lax.dot_general(h2, s_ref[...], (((1,), (0,)), ((), ())),
                            preferred_element_type=jnp.float32)
        out_ref[0] = t
        out_ref[1] = t

    out = pl.pallas_call(
        body,
        grid=(grid,),
        in_specs=[pl.BlockSpec((blk, 8), lambda i: (i, 0)),
                  pl.BlockSpec((blk, 8), lambda i: (grid + i, 0)),
                  pl.BlockSpec((blk, 1), lambda i: (i, 0)),
                  pl.BlockSpec((8, 16), lambda i: (0, 0))],
        out_specs=pl.BlockSpec((2, blk, 16), lambda i: (0, i, 0)),
        out_shape=jax.ShapeDtypeStruct((2, N, 16), jnp.float32),
    )(s2, s2, inv, S2)
    return out.reshape(2 * N, 16)


def _node_final(s3, inv):
    blk = N_BLK
    grid = N // blk

    def body(a_ref, b_ref, inv_ref, out_ref):
        v = (a_ref[...] + b_ref[...]) * inv_ref[...]
        out_ref[...] = v[:, :4]

    return pl.pallas_call(
        body,
        grid=(grid,),
        in_specs=[pl.BlockSpec((blk, 8), lambda i: (i, 0)),
                  pl.BlockSpec((blk, 8), lambda i: (grid + i, 0)),
                  pl.BlockSpec((blk, 1), lambda i: (i, 0))],
        out_specs=pl.BlockSpec((blk, 4), lambda i: (i, 0)),
        out_shape=jax.ShapeDtypeStruct((N, 4), jnp.float32),
    )(s3, s3, inv)


def _pad16(W):
    k, n = W.shape
    return jnp.concatenate([W, jnp.zeros((16 - k, n), W.dtype)], axis=0)


def _padcols(W, width):
    k, n = W.shape
    return jnp.concatenate([W, jnp.zeros((k, width - n), W.dtype)], axis=1)


def kernel(x, edge_index, params):
    src = edge_index[0]
    dst = edge_index[1]
    idx_flat = jnp.concatenate([src, dst + N]).reshape((2 * E) // GRP, GRP)
    dst_grp = dst.reshape(E // GRP, GRP)

    ones_g = jnp.ones((GRP, 8), jnp.float32)
    z8 = jnp.zeros((CHUNK, 8), jnp.float32)
    z16 = jnp.zeros((CHUNK, 16), jnp.float32)

    cnt2 = _sc_scatter_narrow(ones_g, dst_grp, z8, 8, counts_mode=True)
    inv_cnt = _inv_cnt_kernel(
        cnt2[:, 0].reshape(2, 1000, 100)).reshape(N, 1)

    table1 = _bn_table(x.reshape(N // 8, 32), params["bn_gamma"],
                       params["bn_beta"])
    g1 = _sc_gather(table1, idx_flat).reshape(2, E, 16)
    (W1, b1), (W1b2, b1b2), (W1b3, b1b3) = params["enc1"]
    W1a, W1b = W1[:4], W1[4:]
    wl1 = [_pad16(W1b), _pad16(W1a - W1b), b1.reshape(1, -1),
           W1b2, b1b2.reshape(1, -1), W1b3, b1b3.reshape(1, -1)]
    m1 = _edge_mlp(g1, True, True, wl1, 32, split_out=True)
    s1 = _sc_scatter_wide(m1.reshape(2 * E, 16), dst_grp, z16)

    (W2, b2), (W2b2, b2b2), (W2b3, b2b3) = params["enc2"]
    W2a, W2b = W2[:32], W2[32:]
    table2 = _node_enc2_tables(s1, inv_cnt, W2b, W2a - W2b)
    g2 = _sc_gather(table2, idx_flat).reshape(2, E, 16)
    wl2 = [b2.reshape(1, -1), W2b2, b2b2.reshape(1, -1),
           _padcols(W2b3, 8), _padcols(b2b3.reshape(1, -1), 8)]
    m2 = _edge_mlp(g2, False, True, wl2, 8, split_out=False)
    s2 = _sc_scatter_narrow(m2, dst_grp, z8, 8)

    table3 = _node_dec1_table(s2, inv_cnt)
    g3 = _sc_gather(table3, idx_flat).reshape(2, E, 16)
    (W3, b3), (W3b2, b3b2), (W3b3, b3b3) = params["dec1"]
    W3a, W3b = W3[:2], W3[2:]
    wl3 = [_pad16(W3b), _pad16(W3a - W3b), b3.reshape(1, -1),
           W3b2, b3b2.reshape(1, -1),
           _padcols(W3b3, 8), _padcols(b3b3.reshape(1, -1), 8)]
    m3 = _edge_mlp(g3, True, False, wl3, 8, split_out=False)
    s3 = _sc_scatter_narrow(m3, dst_grp, z8, 8)
    return _node_final(s3, inv_cnt)


# speedup vs baseline: 4.2958x; 1.0505x over previous
import functools

import jax
import jax.numpy as jnp
from jax import lax
from jax.experimental import pallas as pl
from jax.experimental.pallas import tpu as pltpu
from jax.experimental.pallas import tpu_sc as plsc

N = 100000
E = 1600000
NC, NS, L = 2, 16, 16
NW = NC * NS

GRP = 100
K_J = 8
CHUNK = GRP * K_J

E_BLK = 10000


def _gather_body(table_hbm, idxg_hbm, out_hbm, idx_v, rows_v, sem):
    wid = lax.axis_index("s") * NC + lax.axis_index("c")
    rows_per_w = (2 * E) // NW
    nchunks = rows_per_w // CHUNK

    half = wid // NS

    def chunk(ci, carry):
        row0 = pl.multiple_of((wid % NS) * rows_per_w + ci * CHUNK, 8)
        grp0 = pl.multiple_of(wid * (rows_per_w // GRP) + ci * K_J, 8)
        pltpu.sync_copy(idxg_hbm.at[pl.ds(grp0, K_J)], idx_v)
        cps = []
        for j in range(K_J):
            cps.append(pltpu.async_copy(
                table_hbm.at[idx_v.at[j]],
                rows_v.at[pl.ds(j * GRP, GRP)], sem))
        for cp in cps:
            cp.wait()
        pltpu.sync_copy(rows_v, out_hbm.at[half, pl.ds(row0, CHUNK)])
        return carry

    lax.fori_loop(0, nchunks, chunk, 0, unroll=False)


def _sc_gather(table, idx_grp):
    mesh = plsc.VectorSubcoreMesh(core_axis_name="c", subcore_axis_name="s")
    f = pl.kernel(
        _gather_body,
        mesh=mesh,
        compiler_params=pltpu.CompilerParams(use_tc_tiling_on_sc=False),
        out_type=jax.ShapeDtypeStruct((2, E, 16), jnp.float32),
        scratch_types=[
            pltpu.VMEM((K_J, GRP), jnp.int32),
            pltpu.VMEM((CHUNK, 16), jnp.float32),
            pltpu.SemaphoreType.DMA,
        ],
    )
    return f(table, idx_grp)


def _zero_acc(zeros_hbm, zb_v, acc, s, width):
    pltpu.sync_copy(zeros_hbm, zb_v)
    nchunks = -(-N // CHUNK)
    for i in range(-(-nchunks // NS)):
        k = i * NS + s
        @pl.when(k * CHUNK < N)
        def _():
            r0 = pl.multiple_of(k * CHUNK, 8)
            pltpu.sync_copy(zb_v, acc.at[pl.ds(r0, CHUNK)])


def _write_out(out_hbm, acc, c, s):
    for i in range(-(-(N // CHUNK) // NS)):
        k = i * NS + s
        @pl.when(k * CHUNK < N)
        def _():
            r0 = pl.multiple_of(k * CHUNK, 8)
            pltpu.sync_copy(acc.at[pl.ds(r0, CHUNK)],
                            out_hbm.at[c, pl.ds(r0, CHUNK)])


def _scatter_wide_body(msgs_hbm, dstg_hbm, zeros_hbm, out_hbm,
                       idx_v, rows_v, zb_v, acc, sem):
    c = lax.axis_index("c")
    s = lax.axis_index("s")
    _zero_acc(zeros_hbm, zb_v, acc, s, 16)
    plsc.subcore_barrier()

    nchunks_per_tile = E // (CHUNK * NS)

    def chunk(i, carry):
        k = i * NS + s
        grp0 = pl.multiple_of(k * K_J, 8)
        row0 = pl.multiple_of(k * CHUNK, 8)
        col0 = pl.multiple_of(c * 16, 8)
        pltpu.sync_copy(dstg_hbm.at[pl.ds(grp0, K_J)], idx_v)
        pltpu.sync_copy(msgs_hbm.at[pl.ds(row0, CHUNK), pl.ds(col0, 16)],
                        rows_v)
        cps = []
        for j in range(K_J):
            cps.append(pltpu.async_copy(
                rows_v.at[pl.ds(j * GRP, GRP)],
                acc.at[idx_v.at[j]], sem, add=True))
        for cp in cps:
            cp.wait()
        return carry

    lax.fori_loop(0, nchunks_per_tile, chunk, 0, unroll=False)
    plsc.subcore_barrier()
    _write_out(out_hbm, acc, c, s)


def _sc_scatter_wide(msgs, dst_grp, zeros_c):
    mesh = plsc.VectorSubcoreMesh(core_axis_name="c", subcore_axis_name="s")
    f = pl.kernel(
        _scatter_wide_body,
        mesh=mesh,
        compiler_params=pltpu.CompilerParams(use_tc_tiling_on_sc=False),
        out_type=jax.ShapeDtypeStruct((2, N, 16), jnp.float32),
        scratch_types=[
            pltpu.VMEM((K_J, GRP), jnp.int32),
            pltpu.VMEM((CHUNK, 16), jnp.float32),
            pltpu.VMEM((CHUNK, 16), jnp.float32),
            pltpu.VMEM_SHARED((N, 16), jnp.float32),
            pltpu.SemaphoreType.DMA,
        ],
    )
    return f(msgs, dst_grp, zeros_c)


def _scatter_narrow_body(F, counts_mode, msgs_hbm, dstg_hbm, zeros_hbm,
                         out_hbm, idx_v, rows_v, zb_v, acc, sem):
    c = lax.axis_index("c")
    s = lax.axis_index("s")
    _zero_acc(zeros_hbm, zb_v, acc, s, F)
    if counts_mode:
        pltpu.sync_copy(msgs_hbm, rows_v)
    plsc.subcore_barrier()

    wid = s * NC + c
    total_chunks = E // CHUNK

    def chunk(i, carry):
        k = i * NW + wid

        @pl.when(k < total_chunks)
        def _():
            grp0 = pl.multiple_of(k * K_J, 8)
            pltpu.sync_copy(dstg_hbm.at[pl.ds(grp0, K_J)], idx_v)
            if not counts_mode:
                row0 = pl.multiple_of(k * CHUNK, 8)
                pltpu.sync_copy(msgs_hbm.at[pl.ds(row0, CHUNK)], rows_v)
            cps = []
            for j in range(K_J):
                src = rows_v if counts_mode else rows_v.at[pl.ds(j * GRP, GRP)]
                cps.append(pltpu.async_copy(
                    src, acc.at[idx_v.at[j]], sem, add=True))
            for cp in cps:
                cp.wait()
        return carry

    lax.fori_loop(0, -(-total_chunks // NW), chunk, 0, unroll=False)
    plsc.subcore_barrier()
    _write_out(out_hbm, acc, c, s)


def _sc_scatter_narrow(msgs, dst_grp, zeros_c, F, counts_mode=False):
    mesh = plsc.VectorSubcoreMesh(core_axis_name="c", subcore_axis_name="s")
    f = pl.kernel(
        functools.partial(_scatter_narrow_body, F, counts_mode),
        mesh=mesh,
        compiler_params=pltpu.CompilerParams(use_tc_tiling_on_sc=False),
        out_type=jax.ShapeDtypeStruct((2, N, F), jnp.float32),
        scratch_types=[
            pltpu.VMEM((K_J, GRP), jnp.int32),
            pltpu.VMEM((GRP, F) if counts_mode else (CHUNK, F), jnp.float32),
            pltpu.VMEM((CHUNK, F), jnp.float32),
            pltpu.VMEM_SHARED((N, F), jnp.float32),
            pltpu.SemaphoreType.DMA,
        ],
    )
    return f(msgs, dst_grp, zeros_c)


def _edge_mlp_body(first_proj, final_relu, src_ref, dst_ref, *rest):
    out_ref = rest[-1]
    wrefs = rest[:-1]
    g_src = src_ref[0]
    g_dst = dst_ref[0]

    def mat(h, w):
        return lax.dot_general(h, w, (((1,), (0,)), ((), ())),
                               preferred_element_type=jnp.float32)

    if first_proj:
        h = mat(g_src, wrefs[0][...]) + mat(g_dst, wrefs[1][...]) + wrefs[2][...]
        i = 3
    else:
        h = g_src + g_dst + wrefs[0][...]
        i = 1
    h = jnp.maximum(h, 0.0)
    n_rest = (len(wrefs) - i) // 2
    for k in range(n_rest):
        h = mat(h, wrefs[i + 2 * k][...]) + wrefs[i + 2 * k + 1][...]
        if k < n_rest - 1 or final_relu:
            h = jnp.maximum(h, 0.0)
    out_ref[...] = h


def _edge_mlp(g, first_proj, final_relu, wlist, f_out):
    grid = E // E_BLK
    w_specs = [pl.BlockSpec(w.shape, lambda i: (0,) * w.ndim) for w in wlist]
    return pl.pallas_call(
        functools.partial(_edge_mlp_body, first_proj, final_relu),
        grid=(grid,),
        in_specs=[pl.BlockSpec((1, E_BLK, 16), lambda i: (0, i, 0)),
                  pl.BlockSpec((1, E_BLK, 16), lambda i: (1, i, 0))] + w_specs,
        out_specs=pl.BlockSpec((E_BLK, f_out), lambda i: (i, 0)),
        out_shape=jax.ShapeDtypeStruct((E, f_out), jnp.float32),
    )(g, g, *wlist)


N_BLK = 5000


def _inv_cnt_kernel(cnt2):
    blk = N_BLK
    grid = N // blk

    def body(a_ref, b_ref, inv_ref):
        c = a_ref[0][:, :1] + b_ref[0][:, :1]
        inv_ref[...] = 1.0 / jnp.maximum(c, 1.0)

    return pl.pallas_call(
        body,
        grid=(grid,),
        in_specs=[pl.BlockSpec((1, blk, 8), lambda i: (0, i, 0)),
                  pl.BlockSpec((1, blk, 8), lambda i: (1, i, 0))],
        out_specs=pl.BlockSpec((blk, 1), lambda i: (i, 0)),
        out_shape=jax.ShapeDtypeStruct((N, 1), jnp.float32),
    )(cnt2, cnt2)


def _bn_table(x32, gamma, beta):
    import numpy as _np
    Gn = _np.zeros((32, 4), _np.float32)
    for i in range(32):
        Gn[i, i % 4] = 1.0
    G = jnp.asarray(Gn)

    def stats_body(x_ref, g_ref, gm_ref, bt_ref, sc_ref, sh_ref):
        xv = x_ref[...]
        g = g_ref[...]
        s = jnp.sum(xv, axis=0, keepdims=True)
        mean4 = lax.dot_general(s, g, (((1,), (0,)), ((), ())),
                                preferred_element_type=jnp.float32) / N
        mean32 = lax.dot_general(mean4, g, (((1,), (1,)), ((), ())),
                                 preferred_element_type=jnp.float32)
        xc = xv - mean32
        v = jnp.sum(xc * xc, axis=0, keepdims=True)
        var4 = lax.dot_general(v, g, (((1,), (0,)), ((), ())),
                               preferred_element_type=jnp.float32) / N
        rs4 = gm_ref[...] * jax.lax.rsqrt(var4 + 1e-5)
        sh4 = bt_ref[...] - mean4 * rs4
        sc_ref[...] = lax.dot_general(rs4, g, (((1,), (1,)), ((), ())),
                                      preferred_element_type=jnp.float32)
        sh_ref[...] = lax.dot_general(sh4, g, (((1,), (1,)), ((), ())),
                                      preferred_element_type=jnp.float32)

    scale32, shift32 = pl.pallas_call(
        stats_body,
        out_shape=(jax.ShapeDtypeStruct((1, 32), jnp.float32),
                   jax.ShapeDtypeStruct((1, 32), jnp.float32)),
    )(x32, G, gamma.reshape(1, 4), beta.reshape(1, 4))

    Sn = _np.zeros((4, 16), _np.float32)
    for f in range(4):
        Sn[f, f] = 1.0
    S = jnp.asarray(Sn)

    blk = N_BLK
    grid = N // blk

    def apply_body(x4_ref, sc4_ref, sh4_ref, s_ref, out_ref):
        h = x4_ref[...] * sc4_ref[...] + sh4_ref[...]
        t = lax.dot_general(h, s_ref[...], (((1,), (0,)), ((), ())),
                            preferred_element_type=jnp.float32)
        out_ref[...] = t

    x4 = x32.reshape(N, 4)
    scale4 = scale32[:, :4]
    shift4 = shift32[:, :4]
    return pl.pallas_call(
        apply_body,
        grid=(2, grid),
        in_specs=[pl.BlockSpec((blk, 4), lambda c, i: (i, 0)),
                  pl.BlockSpec((1, 4), lambda c, i: (0, 0)),
                  pl.BlockSpec((1, 4), lambda c, i: (0, 0)),
                  pl.BlockSpec((4, 16), lambda c, i: (0, 0))],
        out_specs=pl.BlockSpec((blk, 16), lambda c, i: (c * grid + i, 0)),
        out_shape=jax.ShapeDtypeStruct((2 * N, 16), jnp.float32),
    )(x4, scale4, shift4, S)


def _node_enc2_tables(s1, inv, Wsd):
    blk = N_BLK
    grid = N // blk

    def body(a_ref, b_ref, inv_ref, w_ref, out_ref):
        h1 = jnp.concatenate([a_ref[0], b_ref[0]], axis=1) * inv_ref[...]
        out_ref[...] = lax.dot_general(h1, w_ref[0], (((1,), (0,)), ((), ())),
                                      preferred_element_type=jnp.float32)

    return pl.pallas_call(
        body,
        grid=(2, grid),
        in_specs=[pl.BlockSpec((1, blk, 16), lambda c, i: (0, i, 0)),
                  pl.BlockSpec((1, blk, 16), lambda c, i: (1, i, 0)),
                  pl.BlockSpec((blk, 1), lambda c, i: (i, 0)),
                  pl.BlockSpec((1, 32, 16), lambda c, i: (c, 0, 0))],
        out_specs=pl.BlockSpec((blk, 16), lambda c, i: (c * grid + i, 0)),
        out_shape=jax.ShapeDtypeStruct((2 * N, 16), jnp.float32),
    )(s1, s1, inv, Wsd)


def _node_dec1_table(s2, inv):
    blk = N_BLK
    grid = N // blk
    import numpy as _np
    Sn = _np.zeros((8, 16), _np.float32)
    Sn[0, 0] = 1.0
    Sn[1, 1] = 1.0
    S2 = jnp.asarray(Sn)

    def body(a_ref, b_ref, inv_ref, s_ref, out_ref):
        h2 = (a_ref[0] + b_ref[0]) * inv_ref[...]
        out_ref[...] = lax.dot_general(h2, s_ref[...], (((1,), (0,)), ((), ())),
                                      preferred_element_type=jnp.float32)

    return pl.pallas_call(
        body,
        grid=(2, grid),
        in_specs=[pl.BlockSpec((1, blk, 8), lambda c, i: (0, i, 0)),
                  pl.BlockSpec((1, blk, 8), lambda c, i: (1, i, 0)),
                  pl.BlockSpec((blk, 1), lambda c, i: (i, 0)),
                  pl.BlockSpec((8, 16), lambda c, i: (0, 0))],
        out_specs=pl.BlockSpec((blk, 16), lambda c, i: (c * grid + i, 0)),
        out_shape=jax.ShapeDtypeStruct((2 * N, 16), jnp.float32),
    )(s2, s2, inv, S2)


def _node_final(s3, inv):
    blk = N_BLK
    grid = N // blk

    def body(a_ref, b_ref, inv_ref, out_ref):
        v = (a_ref[0] + b_ref[0]) * inv_ref[...]
        out_ref[...] = v[:, :4]

    return pl.pallas_call(
        body,
        grid=(grid,),
        in_specs=[pl.BlockSpec((1, blk, 8), lambda i: (0, i, 0)),
                  pl.BlockSpec((1, blk, 8), lambda i: (1, i, 0)),
                  pl.BlockSpec((blk, 1), lambda i: (i, 0))],
        out_specs=pl.BlockSpec((blk, 4), lambda i: (i, 0)),
        out_shape=jax.ShapeDtypeStruct((N, 4), jnp.float32),
    )(s3, s3, inv)


def _pad16(W):
    k, n = W.shape
    return jnp.concatenate([W, jnp.zeros((16 - k, n), W.dtype)], axis=0)


def _padcols(W, width):
    k, n = W.shape
    return jnp.concatenate([W, jnp.zeros((k, width - n), W.dtype)], axis=1)


def kernel(x, edge_index, params):
    src = edge_index[0]
    dst = edge_index[1]
    idx_flat = jnp.concatenate([src, dst + N]).reshape((2 * E) // GRP, GRP)
    dst_grp = dst.reshape(E // GRP, GRP)

    ones_g = jnp.ones((GRP, 8), jnp.float32)
    z8 = jnp.zeros((CHUNK, 8), jnp.float32)
    z16 = jnp.zeros((CHUNK, 16), jnp.float32)

    cnt2 = _sc_scatter_narrow(ones_g, dst_grp, z8, 8, counts_mode=True)
    inv_cnt = _inv_cnt_kernel(cnt2)

    table1 = _bn_table(x.reshape(N // 8, 32), params["bn_gamma"],
                       params["bn_beta"])
    g1 = _sc_gather(table1, idx_flat)
    (W1, b1), (W1b2, b1b2), (W1b3, b1b3) = params["enc1"]
    W1a, W1b = W1[:4], W1[4:]
    wl1 = [_pad16(W1b), _pad16(W1a - W1b), b1.reshape(1, -1),
           W1b2, b1b2.reshape(1, -1), W1b3, b1b3.reshape(1, -1)]
    m1 = _edge_mlp(g1, True, True, wl1, 32)
    s1 = _sc_scatter_wide(m1, dst_grp, z16)

    (W2, b2), (W2b2, b2b2), (W2b3, b2b3) = params["enc2"]
    W2a, W2b = W2[:32], W2[32:]
    Wsd = jnp.stack([W2b, W2a - W2b])
    table2 = _node_enc2_tables(s1, inv_cnt, Wsd)
    g2 = _sc_gather(table2, idx_flat)
    wl2 = [b2.reshape(1, -1), W2b2, b2b2.reshape(1, -1),
           _padcols(W2b3, 8), _padcols(b2b3.reshape(1, -1), 8)]
    m2 = _edge_mlp(g2, False, True, wl2, 8)
    s2 = _sc_scatter_narrow(m2, dst_grp, z8, 8)

    table3 = _node_dec1_table(s2, inv_cnt)
    g3 = _sc_gather(table3, idx_flat)
    (W3, b3), (W3b2, b3b2), (W3b3, b3b3) = params["dec1"]
    W3a, W3b = W3[:2], W3[2:]
    wl3 = [_pad16(W3b), _pad16(W3a - W3b), b3.reshape(1, -1),
           W3b2, b3b2.reshape(1, -1),
           _padcols(W3b3, 8), _padcols(b3b3.reshape(1, -1), 8)]
    m3 = _edge_mlp(g3, True, False, wl3, 8)
    s3 = _sc_scatter_narrow(m3, dst_grp, z8, 8)
    return _node_final(s3, inv_cnt)


# speedup vs baseline: 14.0531x; 3.2713x over previous
import functools

import jax
import jax.numpy as jnp
import numpy as np
from jax import lax
from jax.experimental import pallas as pl
from jax.experimental.pallas import tpu as pltpu
from jax.experimental.pallas import tpu_sc as plsc

N = 100000
E = 1600000
NC, NS, L = 2, 16, 16
NW = NC * NS

GRP = 100
K_J = 8
CHUNK = GRP * K_J

E8 = E // 8
N8 = N // 8
E_BLK = 2000


def _gather_body(table_hbm, idxg_hbm, out_hbm, idx_v, rows_v, sem):
    wid = lax.axis_index("s") * NC + lax.axis_index("c")
    rows_per_w = (2 * E) // NW
    nchunks = rows_per_w // CHUNK
    half = wid // NS

    def chunk(ci, carry):
        row0 = pl.multiple_of((wid % NS) * rows_per_w + ci * CHUNK, 8)
        grp0 = pl.multiple_of(wid * (rows_per_w // GRP) + ci * K_J, 8)
        pltpu.sync_copy(idxg_hbm.at[pl.ds(grp0, K_J)], idx_v)
        cps = []
        for j in range(K_J):
            cps.append(pltpu.async_copy(
                table_hbm.at[idx_v.at[j]],
                rows_v.at[pl.ds(j * GRP, GRP)], sem))
        for cp in cps:
            cp.wait()
        pltpu.sync_copy(rows_v, out_hbm.at[half, pl.ds(row0, CHUNK)])
        return carry

    lax.fori_loop(0, nchunks, chunk, 0, unroll=False)


def _sc_gather(table, idx_grp):
    mesh = plsc.VectorSubcoreMesh(core_axis_name="c", subcore_axis_name="s")
    f = pl.kernel(
        _gather_body,
        mesh=mesh,
        compiler_params=pltpu.CompilerParams(use_tc_tiling_on_sc=False),
        out_type=jax.ShapeDtypeStruct((2, E, 16), jnp.float32),
        scratch_types=[
            pltpu.VMEM((K_J, GRP), jnp.int32),
            pltpu.VMEM((CHUNK, 16), jnp.float32),
            pltpu.SemaphoreType.DMA,
        ],
    )
    return f(table, idx_grp)


def _zero_acc(zeros_hbm, zb_v, acc, s):
    pltpu.sync_copy(zeros_hbm, zb_v)
    nchunks = -(-N // CHUNK)
    for i in range(-(-nchunks // NS)):
        k = i * NS + s
        @pl.when(k * CHUNK < N)
        def _():
            r0 = pl.multiple_of(k * CHUNK, 8)
            pltpu.sync_copy(zb_v, acc.at[pl.ds(r0, CHUNK)])


def _write_out(out_hbm, acc, c, s):
    for i in range(-(-(N // CHUNK) // NS)):
        k = i * NS + s
        @pl.when(k * CHUNK < N)
        def _():
            r0 = pl.multiple_of(k * CHUNK, 8)
            pltpu.sync_copy(acc.at[pl.ds(r0, CHUNK)],
                            out_hbm.at[c, pl.ds(r0, CHUNK)])


def _scatter_wide_body(msgs_hbm, dstg_hbm, zeros_hbm, out_hbm,
                       idx_v, rows_v, zb_v, acc, sem):
    c = lax.axis_index("c")
    s = lax.axis_index("s")
    _zero_acc(zeros_hbm, zb_v, acc, s)
    plsc.subcore_barrier()

    nchunks_per_tile = E // (CHUNK * NS)

    def chunk(i, carry):
        k = i * NS + s
        grp0 = pl.multiple_of(k * K_J, 8)
        row0 = pl.multiple_of(k * CHUNK, 8)
        col0 = pl.multiple_of(c * 16, 8)
        pltpu.sync_copy(dstg_hbm.at[pl.ds(grp0, K_J)], idx_v)
        pltpu.sync_copy(msgs_hbm.at[pl.ds(row0, CHUNK), pl.ds(col0, 16)],
                        rows_v)
        cps = []
        for j in range(K_J):
            cps.append(pltpu.async_copy(
                rows_v.at[pl.ds(j * GRP, GRP)],
                acc.at[idx_v.at[j]], sem, add=True))
        for cp in cps:
            cp.wait()
        return carry

    lax.fori_loop(0, nchunks_per_tile, chunk, 0, unroll=False)
    plsc.subcore_barrier()
    _write_out(out_hbm, acc, c, s)


def _sc_scatter_wide(msgs, dst_grp, zeros_c):
    mesh = plsc.VectorSubcoreMesh(core_axis_name="c", subcore_axis_name="s")
    f = pl.kernel(
        _scatter_wide_body,
        mesh=mesh,
        compiler_params=pltpu.CompilerParams(use_tc_tiling_on_sc=False),
        out_type=jax.ShapeDtypeStruct((2, N, 16), jnp.float32),
        scratch_types=[
            pltpu.VMEM((K_J, GRP), jnp.int32),
            pltpu.VMEM((CHUNK, 16), jnp.float32),
            pltpu.VMEM((CHUNK, 16), jnp.float32),
            pltpu.VMEM_SHARED((N, 16), jnp.float32),
            pltpu.SemaphoreType.DMA,
        ],
    )
    return f(msgs, dst_grp, zeros_c)


def _scatter16_body(counts_mode, msgs_hbm, dstg_hbm, zeros_hbm,
                    out_hbm, idx_v, rows_v, zb_v, acc, sem):
    c = lax.axis_index("c")
    s = lax.axis_index("s")
    _zero_acc(zeros_hbm, zb_v, acc, s)
    if counts_mode:
        pltpu.sync_copy(msgs_hbm, rows_v)
    plsc.subcore_barrier()

    wid = s * NC + c
    total_chunks = E // CHUNK

    def chunk(i, carry):
        k = i * NW + wid

        @pl.when(k < total_chunks)
        def _():
            grp0 = pl.multiple_of(k * K_J, 8)
            pltpu.sync_copy(dstg_hbm.at[pl.ds(grp0, K_J)], idx_v)
            if not counts_mode:
                row0 = pl.multiple_of(k * CHUNK, 8)
                pltpu.sync_copy(msgs_hbm.at[pl.ds(row0, CHUNK)], rows_v)
            cps = []
            for j in range(K_J):
                src = rows_v if counts_mode else rows_v.at[pl.ds(j * GRP, GRP)]
                cps.append(pltpu.async_copy(
                    src, acc.at[idx_v.at[j]], sem, add=True))
            for cp in cps:
                cp.wait()
        return carry

    lax.fori_loop(0, -(-total_chunks // NW), chunk, 0, unroll=False)
    plsc.subcore_barrier()
    _write_out(out_hbm, acc, c, s)


def _sc_scatter16(msgs, dst_grp, zeros_c, counts_mode=False):
    mesh = plsc.VectorSubcoreMesh(core_axis_name="c", subcore_axis_name="s")
    f = pl.kernel(
        functools.partial(_scatter16_body, counts_mode),
        mesh=mesh,
        compiler_params=pltpu.CompilerParams(use_tc_tiling_on_sc=False),
        out_type=jax.ShapeDtypeStruct((2, N, 16), jnp.float32),
        scratch_types=[
            pltpu.VMEM((K_J, GRP), jnp.int32),
            pltpu.VMEM((GRP, 16) if counts_mode else (CHUNK, 16), jnp.float32),
            pltpu.VMEM((CHUNK, 16), jnp.float32),
            pltpu.VMEM_SHARED((N, 16), jnp.float32),
            pltpu.SemaphoreType.DMA,
        ],
    )
    return f(msgs, dst_grp, zeros_c)


def _edge_mlp_body(first_proj, final_relu, src_ref, dst_ref, *rest):
    out_ref = rest[-1]
    wrefs = rest[:-1]
    g_src = src_ref[0]
    g_dst = dst_ref[0]

    def mat(h, w):
        return lax.dot_general(h, w, (((1,), (0,)), ((), ())),
                               preferred_element_type=jnp.float32)

    if first_proj:
        h = mat(g_src, wrefs[0][...]) + mat(g_dst, wrefs[1][...]) + wrefs[2][...]
        i = 3
    else:
        h = g_src + g_dst + wrefs[0][...]
        i = 1
    h = jnp.maximum(h, 0.0)
    n_rest = (len(wrefs) - i) // 2
    for k in range(n_rest):
        h = mat(h, wrefs[i + 2 * k][...]) + wrefs[i + 2 * k + 1][...]
        if k < n_rest - 1 or final_relu:
            h = jnp.maximum(h, 0.0)
    out_ref[...] = h


def _edge_mlp(g_p, first_proj, final_relu, wlist, w_out):
    grid = E8 // E_BLK
    w_specs = [pl.BlockSpec(w.shape, lambda i: (0,) * w.ndim) for w in wlist]
    return pl.pallas_call(
        functools.partial(_edge_mlp_body, first_proj, final_relu),
        grid=(grid,),
        in_specs=[pl.BlockSpec((1, E_BLK, 128), lambda i: (0, i, 0)),
                  pl.BlockSpec((1, E_BLK, 128), lambda i: (1, i, 0))] + w_specs,
        out_specs=pl.BlockSpec((E_BLK, w_out), lambda i: (i, 0)),
        out_shape=jax.ShapeDtypeStruct((E8, w_out), jnp.float32),
    )(g_p, g_p, *wlist)


def _inv_cnt_kernel(cnt2_p):
    E1 = np.zeros((128, 8), np.float32)
    R8 = np.zeros((8, 128), np.float32)
    for i in range(8):
        E1[16 * i, i] = 1.0
        R8[i, 16 * i:16 * i + 16] = 1.0

    def body(a_ref, e_ref, r_ref, inv_ref):
        c0 = lax.dot_general(a_ref[0] + a_ref[1], e_ref[...],
                             (((1,), (0,)), ((), ())),
                             preferred_element_type=jnp.float32)
        inv8 = 1.0 / jnp.maximum(c0, 1.0)
        inv_ref[...] = lax.dot_general(inv8, r_ref[...],
                                       (((1,), (0,)), ((), ())),
                                       preferred_element_type=jnp.float32)

    return pl.pallas_call(
        body,
        out_shape=jax.ShapeDtypeStruct((N8, 128), jnp.float32),
    )(cnt2_p, jnp.asarray(E1), jnp.asarray(R8))


def _bn_table(x, gamma, beta):
    Gn = np.zeros((32, 4), np.float32)
    for i in range(32):
        Gn[i, i % 4] = 1.0
    G = jnp.asarray(Gn)
    x_p = x.reshape(N8, 32)

    def stats_body(x_ref, g_ref, gm_ref, bt_ref, sc_ref, sh_ref):
        xv = x_ref[...]
        g = g_ref[...]
        s = jnp.sum(xv, axis=0, keepdims=True)
        mean4 = lax.dot_general(s, g, (((1,), (0,)), ((), ())),
                                preferred_element_type=jnp.float32) / N
        mean32 = lax.dot_general(mean4, g, (((1,), (1,)), ((), ())),
                                 preferred_element_type=jnp.float32)
        xc = xv - mean32
        v = jnp.sum(xc * xc, axis=0, keepdims=True)
        var4 = lax.dot_general(v, g, (((1,), (0,)), ((), ())),
                               preferred_element_type=jnp.float32) / N
        rs4 = gm_ref[...] * jax.lax.rsqrt(var4 + 1e-5)
        sh4 = bt_ref[...] - mean4 * rs4
        sc_ref[...] = lax.dot_general(rs4, g, (((1,), (1,)), ((), ())),
                                      preferred_element_type=jnp.float32)
        sh_ref[...] = lax.dot_general(sh4, g, (((1,), (1,)), ((), ())),
                                      preferred_element_type=jnp.float32)

    scale32, shift32 = pl.pallas_call(
        stats_body,
        out_shape=(jax.ShapeDtypeStruct((1, 32), jnp.float32),
                   jax.ShapeDtypeStruct((1, 32), jnp.float32)),
    )(x_p, G, gamma.reshape(1, 4), beta.reshape(1, 4))

    Sn = np.zeros((4, 16), np.float32)
    for f in range(4):
        Sn[f, f] = 1.0
    BDS = jnp.asarray(np.kron(np.eye(8, dtype=np.float32), Sn))

    def apply_body(x_ref, sc_ref, sh_ref, s_ref, out_ref):
        h = x_ref[...] * sc_ref[...] + sh_ref[...]
        out_ref[0] = lax.dot_general(h, s_ref[...], (((1,), (0,)), ((), ())),
                                     preferred_element_type=jnp.float32)

    return pl.pallas_call(
        apply_body,
        grid=(2,),
        in_specs=[pl.BlockSpec((N8, 32), lambda c: (0, 0)),
                  pl.BlockSpec((1, 32), lambda c: (0, 0)),
                  pl.BlockSpec((1, 32), lambda c: (0, 0)),
                  pl.BlockSpec((32, 128), lambda c: (0, 0))],
        out_specs=pl.BlockSpec((1, N8, 128), lambda c: (c, 0, 0)),
        out_shape=jax.ShapeDtypeStruct((2, N8, 128), jnp.float32),
    )(x_p, scale32, shift32, BDS)


def _node_enc2_tables(s1_p, inv_p, BDtop, BDbot):
    def body(s_ref, inv_ref, wt_ref, wb_ref, out_ref):
        ha = s_ref[0] * inv_ref[...]
        hb = s_ref[1] * inv_ref[...]
        out_ref[0] = (
            lax.dot_general(ha, wt_ref[0], (((1,), (0,)), ((), ())),
                            preferred_element_type=jnp.float32)
            + lax.dot_general(hb, wb_ref[0], (((1,), (0,)), ((), ())),
                              preferred_element_type=jnp.float32))

    return pl.pallas_call(
        body,
        grid=(2,),
        in_specs=[pl.BlockSpec((2, N8, 128), lambda c: (0, 0, 0)),
                  pl.BlockSpec((N8, 128), lambda c: (0, 0)),
                  pl.BlockSpec((1, 128, 128), lambda c: (c, 0, 0)),
                  pl.BlockSpec((1, 128, 128), lambda c: (c, 0, 0))],
        out_specs=pl.BlockSpec((1, N8, 128), lambda c: (c, 0, 0)),
        out_shape=jax.ShapeDtypeStruct((2, N8, 128), jnp.float32),
    )(s1_p, inv_p, BDtop, BDbot)


def _node_dec1_table(s2_p, inv_p):
    def body(s_ref, inv_ref, out_ref):
        out_ref[0] = (s_ref[0] + s_ref[1]) * inv_ref[...]

    return pl.pallas_call(
        body,
        grid=(2,),
        in_specs=[pl.BlockSpec((2, N8, 128), lambda c: (0, 0, 0)),
                  pl.BlockSpec((N8, 128), lambda c: (0, 0))],
        out_specs=pl.BlockSpec((1, N8, 128), lambda c: (c, 0, 0)),
        out_shape=jax.ShapeDtypeStruct((2, N8, 128), jnp.float32),
    )(s2_p, inv_p)


def _node_final(s3_p, inv_p):
    def body(s_ref, inv_ref, out_ref):
        out_ref[...] = (s_ref[0] + s_ref[1]) * inv_ref[...]

    return pl.pallas_call(
        body,
        out_shape=jax.ShapeDtypeStruct((N8, 128), jnp.float32),
    )(s3_p, inv_p)


def _pad16(W):
    k, n = W.shape
    return jnp.concatenate([W, jnp.zeros((16 - k, n), W.dtype)], axis=0)


def _padcols(W, width):
    k, n = W.shape
    return jnp.concatenate([W, jnp.zeros((k, width - n), W.dtype)], axis=1)


def _bd(W):
    return jnp.kron(jnp.eye(8, dtype=W.dtype), W)


def _bp(b, width=None):
    if width is not None and b.shape[0] < width:
        b = jnp.concatenate([b, jnp.zeros((width - b.shape[0],), b.dtype)])
    return jnp.tile(b, 8).reshape(1, -1)


def kernel(x, edge_index, params):
    src = edge_index[0]
    dst = edge_index[1]
    idx_flat = jnp.concatenate([src, dst + N]).reshape((2 * E) // GRP, GRP)
    dst_grp = dst.reshape(E // GRP, GRP)

    ones_g = jnp.ones((GRP, 16), jnp.float32)
    z16 = jnp.zeros((CHUNK, 16), jnp.float32)

    cnt2 = _sc_scatter16(ones_g, dst_grp, z16, counts_mode=True)
    inv_p = _inv_cnt_kernel(cnt2.reshape(2, N8, 128))

    table1 = _bn_table(x, params["bn_gamma"], params["bn_beta"])
    g1 = _sc_gather(table1.reshape(2 * N, 16), idx_flat)
    (W1, b1), (W1b2, b1b2), (W1b3, b1b3) = params["enc1"]
    W1a, W1b = W1[:4], W1[4:]
    wl1 = [_bd(_pad16(W1b)), _bd(_pad16(W1a - W1b)), _bp(b1),
           _bd(W1b2), _bp(b1b2), _bd(W1b3), _bp(b1b3)]
    m1 = _edge_mlp(g1.reshape(2, E8, 128), True, True, wl1, 256)
    s1 = _sc_scatter_wide(m1.reshape(E, 32), dst_grp, z16)

    (W2, b2), (W2b2, b2b2), (W2b3, b2b3) = params["enc2"]
    W2a, W2b = W2[:32], W2[32:]
    Wd = W2a - W2b
    BDtop = jnp.stack([_bd(W2b[:16]), _bd(Wd[:16])])
    BDbot = jnp.stack([_bd(W2b[16:]), _bd(Wd[16:])])
    table2 = _node_enc2_tables(s1.reshape(2, N8, 128), inv_p, BDtop, BDbot)
    g2 = _sc_gather(table2.reshape(2 * N, 16), idx_flat)
    wl2 = [_bp(b2), _bd(W2b2), _bp(b2b2),
           _bd(_padcols(W2b3, 16)), _bp(b2b3, 16)]
    m2 = _edge_mlp(g2.reshape(2, E8, 128), False, True, wl2, 128)
    s2 = _sc_scatter16(m2.reshape(E, 16), dst_grp, z16)

    table3 = _node_dec1_table(s2.reshape(2, N8, 128), inv_p)
    g3 = _sc_gather(table3.reshape(2 * N, 16), idx_flat)
    (W3, b3), (W3b2, b3b2), (W3b3, b3b3) = params["dec1"]
    W3a, W3b = W3[:2], W3[2:]
    wl3 = [_bd(_pad16(W3b)), _bd(_pad16(W3a - W3b)), _bp(b3),
           _bd(W3b2), _bp(b3b2),
           _bd(_padcols(W3b3, 16)), _bp(b3b3, 16)]
    m3 = _edge_mlp(g3.reshape(2, E8, 128), True, False, wl3, 128)
    s3 = _sc_scatter16(m3.reshape(E, 16), dst_grp, z16)
    out_p = _node_final(s3.reshape(2, N8, 128), inv_p)
    return out_p.reshape(N, 16)[:, :4]


# speedup vs baseline: 14.9577x; 1.0644x over previous
import functools

import jax
import jax.numpy as jnp
import numpy as np
from jax import lax
from jax.experimental import pallas as pl
from jax.experimental.pallas import tpu as pltpu
from jax.experimental.pallas import tpu_sc as plsc

N = 100000
E = 1600000
NC, NS, L = 2, 16, 16
NW = NC * NS

GRP = 100
K_J = 8
CHUNK = GRP * K_J

E8 = E // 8
N8 = N // 8
E_BLK = 2000


def _gather_body(tsrc_hbm, tdst_hbm, idxg_hbm, out_hbm,
                 idx_v, rows_v0, rows_v1, sem, osem):
    wid = lax.axis_index("s") * NC + lax.axis_index("c")
    rows_per_w = (2 * E) // NW
    nchunks = rows_per_w // CHUNK
    half = wid // NS

    def run(table_hbm):
        def chunk(ci, carry):
            row0 = pl.multiple_of((wid % NS) * rows_per_w + ci * CHUNK, 8)
            grp0 = pl.multiple_of(wid * (rows_per_w // GRP) + ci * K_J, 8)
            pltpu.sync_copy(idxg_hbm.at[pl.ds(grp0, K_J)], idx_v)
            for b, rows_v in ((0, rows_v0), (1, rows_v1)):
                @pl.when(ci % 2 == b)
                def _():
                    @pl.when(ci >= 2)
                    def _():
                        pltpu.make_async_copy(rows_v, out_hbm.at[
                            half, pl.ds(row0, CHUNK)], osem).wait()
                    cps = []
                    for j in range(K_J):
                        cps.append(pltpu.async_copy(
                            table_hbm.at[idx_v.at[j]],
                            rows_v.at[pl.ds(j * GRP, GRP)], sem))
                    for cp in cps:
                        cp.wait()
                    pltpu.async_copy(rows_v,
                                     out_hbm.at[half, pl.ds(row0, CHUNK)],
                                     osem)
            return carry

        lax.fori_loop(0, nchunks, chunk, 0, unroll=False)
        for b, rows_v in ((0, rows_v0), (1, rows_v1)):
            r_last = pl.multiple_of(
                (wid % NS) * rows_per_w + (nchunks - 2 + b) * CHUNK, 8)
            pltpu.make_async_copy(
                rows_v, out_hbm.at[half, pl.ds(r_last, CHUNK)], osem).wait()

    @pl.when(half == 0)
    def _():
        run(tsrc_hbm)

    @pl.when(half == 1)
    def _():
        run(tdst_hbm)


def _sc_gather(table_src, table_dst, idx_grp):
    mesh = plsc.VectorSubcoreMesh(core_axis_name="c", subcore_axis_name="s")
    f = pl.kernel(
        _gather_body,
        mesh=mesh,
        compiler_params=pltpu.CompilerParams(use_tc_tiling_on_sc=False),
        out_type=jax.ShapeDtypeStruct((2, E, 16), jnp.float32),
        scratch_types=[
            pltpu.VMEM((K_J, GRP), jnp.int32),
            pltpu.VMEM((CHUNK, 16), jnp.float32),
            pltpu.VMEM((CHUNK, 16), jnp.float32),
            pltpu.SemaphoreType.DMA,
            pltpu.SemaphoreType.DMA,
        ],
    )
    return f(table_src, table_dst, idx_grp)


def _zero_acc(zeros_hbm, zb_v, acc, s):
    pltpu.sync_copy(zeros_hbm, zb_v)
    nchunks = -(-N // CHUNK)
    for i in range(-(-nchunks // NS)):
        k = i * NS + s
        @pl.when(k * CHUNK < N)
        def _():
            r0 = pl.multiple_of(k * CHUNK, 8)
            pltpu.sync_copy(zb_v, acc.at[pl.ds(r0, CHUNK)])


def _write_out(out_hbm, acc, c, s):
    for i in range(-(-(N // CHUNK) // NS)):
        k = i * NS + s
        @pl.when(k * CHUNK < N)
        def _():
            r0 = pl.multiple_of(k * CHUNK, 8)
            pltpu.sync_copy(acc.at[pl.ds(r0, CHUNK)],
                            out_hbm.at[c, pl.ds(r0, CHUNK)])


def _scatter_wide_body(msgs_hbm, dstg_hbm, zeros_hbm, out_hbm,
                       idx_v, rows_v, zb_v, acc, sem):
    c = lax.axis_index("c")
    s = lax.axis_index("s")
    _zero_acc(zeros_hbm, zb_v, acc, s)
    plsc.subcore_barrier()

    nchunks_per_tile = E // (CHUNK * NS)

    def chunk(i, carry):
        k = i * NS + s
        grp0 = pl.multiple_of(k * K_J, 8)
        row0 = pl.multiple_of(k * CHUNK, 8)
        col0 = pl.multiple_of(c * 16, 8)
        pltpu.sync_copy(dstg_hbm.at[pl.ds(grp0, K_J)], idx_v)
        pltpu.sync_copy(msgs_hbm.at[pl.ds(row0, CHUNK), pl.ds(col0, 16)],
                        rows_v)
        cps = []
        for j in range(K_J):
            cps.append(pltpu.async_copy(
                rows_v.at[pl.ds(j * GRP, GRP)],
                acc.at[idx_v.at[j]], sem, add=True))
        for cp in cps:
            cp.wait()
        return carry

    lax.fori_loop(0, nchunks_per_tile, chunk, 0, unroll=False)
    plsc.subcore_barrier()
    _write_out(out_hbm, acc, c, s)


def _sc_scatter_wide(msgs, dst_grp, zeros_c):
    mesh = plsc.VectorSubcoreMesh(core_axis_name="c", subcore_axis_name="s")
    f = pl.kernel(
        _scatter_wide_body,
        mesh=mesh,
        compiler_params=pltpu.CompilerParams(use_tc_tiling_on_sc=False),
        out_type=jax.ShapeDtypeStruct((2, N, 16), jnp.float32),
        scratch_types=[
            pltpu.VMEM((K_J, GRP), jnp.int32),
            pltpu.VMEM((CHUNK, 16), jnp.float32),
            pltpu.VMEM((CHUNK, 16), jnp.float32),
            pltpu.VMEM_SHARED((N, 16), jnp.float32),
            pltpu.SemaphoreType.DMA,
        ],
    )
    return f(msgs, dst_grp, zeros_c)


def _scatter16_body(counts_mode, msgs_hbm, dstg_hbm, zeros_hbm,
                    out_hbm, idx_v, rows_v, zb_v, acc, sem):
    c = lax.axis_index("c")
    s = lax.axis_index("s")
    _zero_acc(zeros_hbm, zb_v, acc, s)
    if counts_mode:
        pltpu.sync_copy(msgs_hbm, rows_v)
    plsc.subcore_barrier()

    wid = s * NC + c
    total_chunks = E // CHUNK

    def chunk(i, carry):
        k = i * NW + wid

        @pl.when(k < total_chunks)
        def _():
            grp0 = pl.multiple_of(k * K_J, 8)
            pltpu.sync_copy(dstg_hbm.at[pl.ds(grp0, K_J)], idx_v)
            if not counts_mode:
                row0 = pl.multiple_of(k * CHUNK, 8)
                pltpu.sync_copy(msgs_hbm.at[pl.ds(row0, CHUNK)], rows_v)
            cps = []
            for j in range(K_J):
                src = rows_v if counts_mode else rows_v.at[pl.ds(j * GRP, GRP)]
                cps.append(pltpu.async_copy(
                    src, acc.at[idx_v.at[j]], sem, add=True))
            for cp in cps:
                cp.wait()
        return carry

    lax.fori_loop(0, -(-total_chunks // NW), chunk, 0, unroll=False)
    plsc.subcore_barrier()
    _write_out(out_hbm, acc, c, s)


def _sc_scatter16(msgs, dst_grp, zeros_c, counts_mode=False):
    mesh = plsc.VectorSubcoreMesh(core_axis_name="c", subcore_axis_name="s")
    f = pl.kernel(
        functools.partial(_scatter16_body, counts_mode),
        mesh=mesh,
        compiler_params=pltpu.CompilerParams(use_tc_tiling_on_sc=False),
        out_type=jax.ShapeDtypeStruct((2, N, 16), jnp.float32),
        scratch_types=[
            pltpu.VMEM((K_J, GRP), jnp.int32),
            pltpu.VMEM((GRP, 16) if counts_mode else (CHUNK, 16), jnp.float32),
            pltpu.VMEM((CHUNK, 16), jnp.float32),
            pltpu.VMEM_SHARED((N, 16), jnp.float32),
            pltpu.SemaphoreType.DMA,
        ],
    )
    return f(msgs, dst_grp, zeros_c)


def _edge_mlp_body(first_proj, final_relu, src_ref, dst_ref, *rest):
    out_ref = rest[-1]
    wrefs = rest[:-1]
    g_src = src_ref[0]
    g_dst = dst_ref[0]

    def mat(h, w):
        return lax.dot_general(h, w, (((1,), (0,)), ((), ())),
                               preferred_element_type=jnp.float32)

    if first_proj:
        h = mat(g_src, wrefs[0][...]) + mat(g_dst, wrefs[1][...]) + wrefs[2][...]
        i = 3
    else:
        h = g_src + g_dst + wrefs[0][...]
        i = 1
    h = jnp.maximum(h, 0.0)
    n_rest = (len(wrefs) - i) // 2
    for k in range(n_rest):
        h = mat(h, wrefs[i + 2 * k][...]) + wrefs[i + 2 * k + 1][...]
        if k < n_rest - 1 or final_relu:
            h = jnp.maximum(h, 0.0)
    out_ref[...] = h


def _edge_mlp(g_p, first_proj, final_relu, wlist, w_out):
    grid = E8 // E_BLK
    w_specs = [pl.BlockSpec(w.shape, lambda i: (0,) * w.ndim) for w in wlist]
    return pl.pallas_call(
        functools.partial(_edge_mlp_body, first_proj, final_relu),
        grid=(grid,),
        in_specs=[pl.BlockSpec((1, E_BLK, 128), lambda i: (0, i, 0)),
                  pl.BlockSpec((1, E_BLK, 128), lambda i: (1, i, 0))] + w_specs,
        out_specs=pl.BlockSpec((E_BLK, w_out), lambda i: (i, 0)),
        out_shape=jax.ShapeDtypeStruct((E8, w_out), jnp.float32),
    )(g_p, g_p, *wlist)


def _inv_cnt_kernel(cnt2_p):
    E1 = np.zeros((128, 8), np.float32)
    R8 = np.zeros((8, 128), np.float32)
    for i in range(8):
        E1[16 * i, i] = 1.0
        R8[i, 16 * i:16 * i + 16] = 1.0

    def body(a_ref, e_ref, r_ref, inv_ref):
        c0 = lax.dot_general(a_ref[0] + a_ref[1], e_ref[...],
                             (((1,), (0,)), ((), ())),
                             preferred_element_type=jnp.float32)
        inv8 = 1.0 / jnp.maximum(c0, 1.0)
        inv_ref[...] = lax.dot_general(inv8, r_ref[...],
                                       (((1,), (0,)), ((), ())),
                                       preferred_element_type=jnp.float32)

    return pl.pallas_call(
        body,
        out_shape=jax.ShapeDtypeStruct((N8, 128), jnp.float32),
    )(cnt2_p, jnp.asarray(E1), jnp.asarray(R8))


def _bn_table(x, gamma, beta):
    Gn = np.zeros((32, 4), np.float32)
    for i in range(32):
        Gn[i, i % 4] = 1.0
    G = jnp.asarray(Gn)
    x_p = x.reshape(N8, 32)

    def stats_body(x_ref, g_ref, gm_ref, bt_ref, sc_ref, sh_ref):
        xv = x_ref[...]
        g = g_ref[...]
        s = jnp.sum(xv, axis=0, keepdims=True)
        mean4 = lax.dot_general(s, g, (((1,), (0,)), ((), ())),
                                preferred_element_type=jnp.float32) / N
        mean32 = lax.dot_general(mean4, g, (((1,), (1,)), ((), ())),
                                 preferred_element_type=jnp.float32)
        xc = xv - mean32
        v = jnp.sum(xc * xc, axis=0, keepdims=True)
        var4 = lax.dot_general(v, g, (((1,), (0,)), ((), ())),
                               preferred_element_type=jnp.float32) / N
        rs4 = gm_ref[...] * jax.lax.rsqrt(var4 + 1e-5)
        sh4 = bt_ref[...] - mean4 * rs4
        sc_ref[...] = lax.dot_general(rs4, g, (((1,), (1,)), ((), ())),
                                      preferred_element_type=jnp.float32)
        sh_ref[...] = lax.dot_general(sh4, g, (((1,), (1,)), ((), ())),
                                      preferred_element_type=jnp.float32)

    scale32, shift32 = pl.pallas_call(
        stats_body,
        out_shape=(jax.ShapeDtypeStruct((1, 32), jnp.float32),
                   jax.ShapeDtypeStruct((1, 32), jnp.float32)),
    )(x_p, G, gamma.reshape(1, 4), beta.reshape(1, 4))

    Sn = np.zeros((4, 16), np.float32)
    for f in range(4):
        Sn[f, f] = 1.0
    BDS = jnp.asarray(np.kron(np.eye(8, dtype=np.float32), Sn))

    def apply_body(x_ref, sc_ref, sh_ref, s_ref, out_ref):
        h = x_ref[...] * sc_ref[...] + sh_ref[...]
        out_ref[...] = lax.dot_general(h, s_ref[...], (((1,), (0,)), ((), ())),
                                       preferred_element_type=jnp.float32)

    return pl.pallas_call(
        apply_body,
        out_shape=jax.ShapeDtypeStruct((N8, 128), jnp.float32),
    )(x_p, scale32, shift32, BDS)


def _node_enc2_tables(s1_p, inv_p, BDtop, BDbot):
    def body(s_ref, inv_ref, wt_ref, wb_ref, out_ref):
        ha = s_ref[0] * inv_ref[...]
        hb = s_ref[1] * inv_ref[...]
        out_ref[0] = (
            lax.dot_general(ha, wt_ref[0], (((1,), (0,)), ((), ())),
                            preferred_element_type=jnp.float32)
            + lax.dot_general(hb, wb_ref[0], (((1,), (0,)), ((), ())),
                              preferred_element_type=jnp.float32))

    return pl.pallas_call(
        body,
        grid=(2,),
        in_specs=[pl.BlockSpec((2, N8, 128), lambda c: (0, 0, 0)),
                  pl.BlockSpec((N8, 128), lambda c: (0, 0)),
                  pl.BlockSpec((1, 128, 128), lambda c: (c, 0, 0)),
                  pl.BlockSpec((1, 128, 128), lambda c: (c, 0, 0))],
        out_specs=pl.BlockSpec((1, N8, 128), lambda c: (c, 0, 0)),
        out_shape=jax.ShapeDtypeStruct((2, N8, 128), jnp.float32),
    )(s1_p, inv_p, BDtop, BDbot)


def _node_dec1_table(s2_p, inv_p):
    def body(s_ref, inv_ref, out_ref):
        out_ref[...] = (s_ref[0] + s_ref[1]) * inv_ref[...]

    return pl.pallas_call(
        body,
        out_shape=jax.ShapeDtypeStruct((N8, 128), jnp.float32),
    )(s2_p, inv_p)


def _node_final(s3_p, inv_p):
    def body(s_ref, inv_ref, out_ref):
        out_ref[...] = (s_ref[0] + s_ref[1]) * inv_ref[...]

    return pl.pallas_call(
        body,
        out_shape=jax.ShapeDtypeStruct((N8, 128), jnp.float32),
    )(s3_p, inv_p)


def _pad16(W):
    k, n = W.shape
    return jnp.concatenate([W, jnp.zeros((16 - k, n), W.dtype)], axis=0)


def _padcols(W, width):
    k, n = W.shape
    return jnp.concatenate([W, jnp.zeros((k, width - n), W.dtype)], axis=1)


def _bd(W):
    return jnp.kron(jnp.eye(8, dtype=W.dtype), W)


def _bp(b, width=None):
    if width is not None and b.shape[0] < width:
        b = jnp.concatenate([b, jnp.zeros((width - b.shape[0],), b.dtype)])
    return jnp.tile(b, 8).reshape(1, -1)


def kernel(x, edge_index, params):
    dst = edge_index[1]
    idx_grp = edge_index.reshape((2 * E) // GRP, GRP)
    dst_grp = dst.reshape(E // GRP, GRP)

    ones_g = jnp.ones((GRP, 16), jnp.float32)
    z16 = jnp.zeros((CHUNK, 16), jnp.float32)

    cnt2 = _sc_scatter16(ones_g, dst_grp, z16, counts_mode=True)
    inv_p = _inv_cnt_kernel(cnt2.reshape(2, N8, 128))

    t1 = _bn_table(x, params["bn_gamma"], params["bn_beta"]).reshape(N, 16)
    g1 = _sc_gather(t1, t1, idx_grp)
    (W1, b1), (W1b2, b1b2), (W1b3, b1b3) = params["enc1"]
    W1a, W1b = W1[:4], W1[4:]
    wl1 = [_bd(_pad16(W1b)), _bd(_pad16(W1a - W1b)), _bp(b1),
           _bd(W1b2), _bp(b1b2), _bd(W1b3), _bp(b1b3)]
    m1 = _edge_mlp(g1.reshape(2, E8, 128), True, True, wl1, 256)
    s1 = _sc_scatter_wide(m1.reshape(E, 32), dst_grp, z16)

    (W2, b2), (W2b2, b2b2), (W2b3, b2b3) = params["enc2"]
    W2a, W2b = W2[:32], W2[32:]
    Wd = W2a - W2b
    BDtop = jnp.stack([_bd(W2b[:16]), _bd(Wd[:16])])
    BDbot = jnp.stack([_bd(W2b[16:]), _bd(Wd[16:])])
    table2 = _node_enc2_tables(s1.reshape(2, N8, 128), inv_p, BDtop, BDbot)
    g2 = _sc_gather(table2[0].reshape(N, 16), table2[1].reshape(N, 16),
                    idx_grp)
    wl2 = [_bp(b2), _bd(W2b2), _bp(b2b2),
           _bd(_padcols(W2b3, 16)), _bp(b2b3, 16)]
    m2 = _edge_mlp(g2.reshape(2, E8, 128), False, True, wl2, 128)
    s2 = _sc_scatter16(m2.reshape(E, 16), dst_grp, z16)

    t3 = _node_dec1_table(s2.reshape(2, N8, 128), inv_p).reshape(N, 16)
    g3 = _sc_gather(t3, t3, idx_grp)
    (W3, b3), (W3b2, b3b2), (W3b3, b3b3) = params["dec1"]
    W3a, W3b = W3[:2], W3[2:]
    wl3 = [_bd(_pad16(W3b)), _bd(_pad16(W3a - W3b)), _bp(b3),
           _bd(W3b2), _bp(b3b2),
           _bd(_padcols(W3b3, 16)), _bp(b3b3, 16)]
    m3 = _edge_mlp(g3.reshape(2, E8, 128), True, False, wl3, 128)
    s3 = _sc_scatter16(m3.reshape(E, 16), dst_grp, z16)
    out_p = _node_final(s3.reshape(2, N8, 128), inv_p)
    return out_p.reshape(N, 16)[:, :4]
